# baseline (speedup 1.0000x reference)
import functools

import jax
import jax.numpy as jnp
from jax import lax
from jax.experimental import pallas as pl
from jax.experimental.pallas import tpu as pltpu
from jax.experimental.pallas import tpu_sc as plsc

N_NODES = 10000
N_PAD = 10240
N_EDGES = 320000
F_IN = 128
F_HID = 16
F_EMB = 32

NC = 2
NS = 16
CH = 128
NCHUNK = N_EDGES // CH
CPW = NCHUNK // (NC * NS)
NTAIL = NCHUNK - CPW * NC * NS
NB = 4
RPD = N_PAD // NS
RPA = N_NODES // NS

_mesh = plsc.VectorSubcoreMesh(core_axis_name="c", subcore_axis_name="s")
_sc_params = pltpu.CompilerParams(use_tc_tiling_on_sc=False,
                                  disable_bounds_checks=True)


@functools.partial(
    pl.kernel,
    out_type=jax.ShapeDtypeStruct((NC, N_PAD), jnp.float32),
    mesh=_mesh,
    compiler_params=_sc_params,
    scratch_types=[
        pltpu.VMEM((CPW + 1, CH), jnp.int32),
        pltpu.VMEM((CH,), jnp.float32),
        pltpu.VMEM_SHARED((N_PAD,), jnp.float32),
        pltpu.SemaphoreType.DMA,
    ],
)
def _sc_deg(dst_hbm, zeros_hbm, out_hbm, dst_v, ones_v, acc, sem):
    c = lax.axis_index("c")
    s = lax.axis_index("s")
    w = c * NS + s
    for k in range(CH // 16):
        ones_v[pl.ds(k * 16, 16)] = jnp.ones((16,), jnp.float32)
    pltpu.sync_copy(zeros_hbm.at[pl.ds(s * RPD, RPD)], acc.at[pl.ds(s * RPD, RPD)])
    pltpu.sync_copy(dst_hbm.at[pl.ds(w * CPW, CPW)], dst_v.at[pl.ds(0, CPW)])

    @pl.when(w < NTAIL)
    def _():
        pltpu.sync_copy(dst_hbm.at[pl.ds(NC * NS * CPW + w, 1)],
                        dst_v.at[pl.ds(CPW, 1)])

    plsc.subcore_barrier()

    def fire(i, carry):
        pltpu.async_copy(ones_v, acc.at[dst_v.at[i]], sem, add=True)
        return carry

    def drain(i, carry):
        pltpu.make_async_copy(ones_v, acc.at[dst_v.at[i]], sem).wait()
        return carry

    n = CPW + jnp.where(w < NTAIL, 1, 0)
    lax.fori_loop(0, n, fire, 0)
    lax.fori_loop(0, n, drain, 0)
    plsc.subcore_barrier()
    pltpu.sync_copy(acc.at[pl.ds(s * RPD, RPD)], out_hbm.at[c, pl.ds(s * RPD, RPD)])


def _agg_pipeline(tab, acc, src_v, dst_v, rows_v, gsem, ssem, w):
    def g_start(i, b):
        pltpu.async_copy(tab.at[src_v.at[i]], rows_v.at[b], gsem.at[b])

    def g_wait(i, b):
        pltpu.make_async_copy(tab.at[src_v.at[i]], rows_v.at[b],
                              gsem.at[b]).wait()

    def s_start(i, b):
        pltpu.async_copy(rows_v.at[b], acc.at[dst_v.at[i]], ssem.at[b],
                         add=True)

    def s_wait(i, b):
        pltpu.make_async_copy(rows_v.at[b], acc.at[dst_v.at[i]],
                              ssem.at[b]).wait()

    g_start(0, 0)
    g_start(1, 1)

    def chunk(i, carry):
        b = lax.rem(i, NB)
        g_wait(i, b)
        s_start(i, b)

        @pl.when(i >= 2)
        def _():
            s_wait(i - 2, lax.rem(i - 2, NB))

        @pl.when(i + 2 < CPW)
        def _():
            g_start(i + 2, lax.rem(i + 2, NB))

        return carry

    lax.fori_loop(0, CPW, chunk, 0)
    s_wait(CPW - 2, (CPW - 2) % NB)
    s_wait(CPW - 1, (CPW - 1) % NB)

    @pl.when(w < NTAIL)
    def _():
        pltpu.async_copy(tab.at[src_v.at[CPW]], rows_v.at[0], gsem.at[0]).wait()
        pltpu.sync_copy(rows_v.at[0], acc.at[dst_v.at[CPW]], add=True)


def _load_idx(src_hbm, dst_hbm, src_v, dst_v, w):
    pltpu.sync_copy(src_hbm.at[pl.ds(w * CPW, CPW)], src_v.at[pl.ds(0, CPW)])
    pltpu.sync_copy(dst_hbm.at[pl.ds(w * CPW, CPW)], dst_v.at[pl.ds(0, CPW)])

    @pl.when(w < NTAIL)
    def _():
        pltpu.sync_copy(src_hbm.at[pl.ds(NC * NS * CPW + w, 1)],
                        src_v.at[pl.ds(CPW, 1)])
        pltpu.sync_copy(dst_hbm.at[pl.ds(NC * NS * CPW + w, 1)],
                        dst_v.at[pl.ds(CPW, 1)])


@functools.partial(
    pl.kernel,
    out_type=jax.ShapeDtypeStruct((NC, N_NODES, F_HID), jnp.float32),
    mesh=_mesh,
    compiler_params=_sc_params,
    scratch_types=[
        pltpu.VMEM((CPW + 1, CH), jnp.int32),
        pltpu.VMEM((CPW + 1, CH), jnp.int32),
        pltpu.VMEM((NB, CH, F_HID), jnp.float32),
        pltpu.VMEM_SHARED((N_NODES, F_HID), jnp.float32),
        pltpu.SemaphoreType.DMA((NB,)),
        pltpu.SemaphoreType.DMA((NB,)),
    ],
)
def _sc_agg16(tab_hbm, src_hbm, dst_hbm, zeros_hbm, out_hbm,
              src_v, dst_v, rows_v, acc, gsem, ssem):
    c = lax.axis_index("c")
    s = lax.axis_index("s")
    w = c * NS + s
    rows = pl.ds(s * RPA, RPA)
    pltpu.sync_copy(zeros_hbm.at[rows], acc.at[rows])
    _load_idx(src_hbm, dst_hbm, src_v, dst_v, w)
    plsc.subcore_barrier()
    _agg_pipeline(tab_hbm, acc, src_v, dst_v, rows_v, gsem, ssem, w)
    plsc.subcore_barrier()
    pltpu.sync_copy(acc.at[rows], out_hbm.at[c, rows])


@functools.partial(
    pl.kernel,
    out_type=(
        jax.ShapeDtypeStruct((NC, N_NODES, F_HID), jnp.float32),
        jax.ShapeDtypeStruct((NC, N_NODES, F_HID), jnp.float32),
    ),
    mesh=_mesh,
    compiler_params=_sc_params,
    scratch_types=[
        pltpu.VMEM((RPA, F_HID), jnp.float32),
        pltpu.VMEM((RPA, F_HID), jnp.float32),
        pltpu.VMEM((RPA, F_HID), jnp.float32),
        pltpu.VMEM((RPA, F_HID), jnp.float32),
        pltpu.VMEM((F_HID,), jnp.float32),
        pltpu.VMEM((CPW + 1, CH), jnp.int32),
        pltpu.VMEM((CPW + 1, CH), jnp.int32),
        pltpu.VMEM((NB, CH, F_HID), jnp.float32),
        pltpu.VMEM_SHARED((N_NODES, F_HID), jnp.float32),
        pltpu.SemaphoreType.DMA((NB,)),
        pltpu.SemaphoreType.DMA((NB,)),
    ],
)
def _sc_agg_fused(h1p_hbm, p1_hbm, dinv16_hbm, b1_hbm, src_hbm, dst_hbm,
                  zeros_hbm, y_hbm, out_hbm,
                  hb, pa, pb, dv, b1v, src_v, dst_v, rows_v, acc, gsem, ssem):
    c = lax.axis_index("c")
    s = lax.axis_index("s")
    w = c * NS + s
    rows = pl.ds(s * RPA, RPA)
    pltpu.sync_copy(zeros_hbm.at[rows], acc.at[rows])
    pltpu.sync_copy(h1p_hbm.at[rows], hb)
    pltpu.sync_copy(p1_hbm.at[0, rows], pa)
    pltpu.sync_copy(p1_hbm.at[1, rows], pb)
    pltpu.sync_copy(dinv16_hbm.at[rows], dv)
    pltpu.sync_copy(b1_hbm, b1v)
    _load_idx(src_hbm, dst_hbm, src_v, dst_v, w)
    bv = b1v[...]

    def yrow(i, carry):
        t = (hb[i, :] + pa[i, :] + pb[i, :]) * dv[i, :] + bv
        hb[i, :] = jnp.maximum(t, 0.0) * dv[i, :]
        return carry

    lax.fori_loop(0, RPA, yrow, 0)
    pltpu.sync_copy(hb, y_hbm.at[c, rows])
    plsc.subcore_barrier()
    _agg_pipeline(y_hbm.at[c], acc, src_v, dst_v, rows_v, gsem, ssem, w)
    plsc.subcore_barrier()
    pltpu.sync_copy(acc.at[rows], out_hbm.at[c, rows])


_BLK = 1000
_GRID = N_NODES // _BLK


def _tc1a_body(x_ref, w1_ref, h1_ref):
    h1_ref[...] = jnp.dot(x_ref[...], w1_ref[...],
                          preferred_element_type=jnp.float32)


def _tc1a(x, W1):
    return pl.pallas_call(
        _tc1a_body,
        grid=(_GRID,),
        in_specs=[
            pl.BlockSpec((_BLK, F_IN), lambda i: (i, 0)),
            pl.BlockSpec((F_IN, F_HID), lambda i: (0, 0)),
        ],
        out_specs=pl.BlockSpec((_BLK, F_HID), lambda i: (i, 0)),
        out_shape=jax.ShapeDtypeStruct((N_NODES, F_HID), jnp.float32),
    )(x, W1)


def _tc1b_body(h1_ref, pd0_ref, pd1_ref, h1p_ref, dinv_ref, dinv16_ref):
    deg = pd0_ref[...] + pd1_ref[...] + 1.0
    dinv = lax.rsqrt(deg)
    h1p_ref[...] = h1_ref[...] * dinv
    dinv_ref[...] = dinv
    dinv16_ref[...] = jnp.broadcast_to(dinv, (_BLK, F_HID))


def _tc1b(h1, pd0, pd1):
    return pl.pallas_call(
        _tc1b_body,
        grid=(_GRID,),
        in_specs=[
            pl.BlockSpec((_BLK, F_HID), lambda i: (i, 0)),
            pl.BlockSpec((_BLK, 1), lambda i: (i, 0)),
            pl.BlockSpec((_BLK, 1), lambda i: (i, 0)),
        ],
        out_specs=[
            pl.BlockSpec((_BLK, F_HID), lambda i: (i, 0)),
            pl.BlockSpec((_BLK, 1), lambda i: (i, 0)),
            pl.BlockSpec((_BLK, F_HID), lambda i: (i, 0)),
        ],
        out_shape=[
            jax.ShapeDtypeStruct((N_NODES, F_HID), jnp.float32),
            jax.ShapeDtypeStruct((N_NODES, 1), jnp.float32),
            jax.ShapeDtypeStruct((N_NODES, F_HID), jnp.float32),
        ],
    )(h1, pd0, pd1)


def _tc3_body(y_ref, p0_ref, p1_ref, dinv_ref, b2_ref, w2_ref, out_ref):
    t = y_ref[0] + p0_ref[0] + p1_ref[0]
    h2 = jnp.dot(t, w2_ref[...], preferred_element_type=jnp.float32)
    out_ref[...] = h2 * dinv_ref[...] + b2_ref[...]


def _tc3(y2, p2, dinv, b2, W2):
    return pl.pallas_call(
        _tc3_body,
        grid=(_GRID,),
        in_specs=[
            pl.BlockSpec((1, _BLK, F_HID), lambda i: (0, i, 0)),
            pl.BlockSpec((1, _BLK, F_HID), lambda i: (0, i, 0)),
            pl.BlockSpec((1, _BLK, F_HID), lambda i: (1, i, 0)),
            pl.BlockSpec((_BLK, 1), lambda i: (i, 0)),
            pl.BlockSpec((1, F_EMB), lambda i: (0, 0)),
            pl.BlockSpec((F_HID, F_EMB), lambda i: (0, 0)),
        ],
        out_specs=pl.BlockSpec((_BLK, F_EMB), lambda i: (i, 0)),
        out_shape=jax.ShapeDtypeStruct((N_NODES, F_EMB), jnp.float32),
    )(y2, p2, p2, dinv, b2, W2)


def kernel(x, edge_index, W1, b1, W2, b2):
    src = edge_index[0].reshape(NCHUNK, CH)
    dst = edge_index[1].reshape(NCHUNK, CH)
    z1 = jnp.zeros((N_PAD,), jnp.float32)
    z16 = jnp.zeros((N_NODES, F_HID), jnp.float32)

    h1 = _tc1a(x, W1)
    pd = _sc_deg(dst, z1)
    h1p, dinv, dinv16 = _tc1b(h1, pd[0][:, None], pd[1][:, None])
    p1 = _sc_agg16(h1p, src, dst, z16)
    y2, p2 = _sc_agg_fused(h1p, p1, dinv16, b1, src, dst, z16)
    return _tc3(y2, p2, dinv, b2.reshape(1, F_EMB), W2)


# speedup vs baseline: 1.1016x; 1.1016x over previous
import functools

import jax
import jax.numpy as jnp
from jax import lax
from jax.experimental import pallas as pl
from jax.experimental.pallas import tpu as pltpu
from jax.experimental.pallas import tpu_sc as plsc

N_NODES = 10000
N_PAD = 10240
N_EDGES = 320000
F_IN = 128
F_HID = 16
F_EMB = 32

NC = 2
NS = 16
CH = 128
NCHUNK = N_EDGES // CH
CPW = NCHUNK // (NC * NS)
NTAIL = NCHUNK - CPW * NC * NS
NB = 6
LAG = 3
RPD = N_PAD // NS
RPA = N_NODES // NS

_mesh = plsc.VectorSubcoreMesh(core_axis_name="c", subcore_axis_name="s")
_sc_params = pltpu.CompilerParams(use_tc_tiling_on_sc=False)


@functools.partial(
    pl.kernel,
    out_type=jax.ShapeDtypeStruct((NC, N_PAD), jnp.float32),
    mesh=_mesh,
    compiler_params=_sc_params,
    scratch_types=[
        pltpu.VMEM((CPW + 1, CH), jnp.int32),
        pltpu.VMEM((CH,), jnp.float32),
        pltpu.VMEM_SHARED((N_PAD,), jnp.float32),
        pltpu.SemaphoreType.DMA,
    ],
)
def _sc_deg(dst_hbm, zeros_hbm, out_hbm, dst_v, ones_v, acc, sem):
    c = lax.axis_index("c")
    s = lax.axis_index("s")
    w = c * NS + s
    for k in range(CH // 16):
        ones_v[pl.ds(k * 16, 16)] = jnp.ones((16,), jnp.float32)
    pltpu.sync_copy(zeros_hbm.at[pl.ds(s * RPD, RPD)], acc.at[pl.ds(s * RPD, RPD)])
    pltpu.sync_copy(dst_hbm.at[pl.ds(w * CPW, CPW)], dst_v.at[pl.ds(0, CPW)])

    @pl.when(w < NTAIL)
    def _():
        pltpu.sync_copy(dst_hbm.at[pl.ds(NC * NS * CPW + w, 1)],
                        dst_v.at[pl.ds(CPW, 1)])

    plsc.subcore_barrier()

    def fire(i, carry):
        pltpu.async_copy(ones_v, acc.at[dst_v.at[i]], sem, add=True)
        return carry

    def drain(i, carry):
        pltpu.make_async_copy(ones_v, acc.at[dst_v.at[i]], sem).wait()
        return carry

    n = CPW + jnp.where(w < NTAIL, 1, 0)
    lax.fori_loop(0, n, fire, 0)
    lax.fori_loop(0, n, drain, 0)
    plsc.subcore_barrier()
    pltpu.sync_copy(acc.at[pl.ds(s * RPD, RPD)], out_hbm.at[c, pl.ds(s * RPD, RPD)])


def _agg_pipeline(tab, acc, src_v, dst_v, rows_v, gsem, ssem, w):
    def g_start(i, b):
        pltpu.async_copy(tab.at[src_v.at[i]], rows_v.at[b], gsem.at[b])

    def g_wait(i, b):
        pltpu.make_async_copy(tab.at[src_v.at[i]], rows_v.at[b],
                              gsem.at[b]).wait()

    def s_start(i, b):
        pltpu.async_copy(rows_v.at[b], acc.at[dst_v.at[i]], ssem.at[b],
                         add=True)

    def s_wait(i, b):
        pltpu.make_async_copy(rows_v.at[b], acc.at[dst_v.at[i]],
                              ssem.at[b]).wait()

    for j in range(LAG):
        g_start(j, j)

    def chunk(i, carry):
        b = lax.rem(i, NB)
        g_wait(i, b)
        s_start(i, b)

        @pl.when(i >= LAG)
        def _():
            s_wait(i - LAG, lax.rem(i - LAG, NB))

        @pl.when(i + LAG < CPW)
        def _():
            g_start(i + LAG, lax.rem(i + LAG, NB))

        return carry

    lax.fori_loop(0, CPW, chunk, 0)
    for j in range(LAG):
        s_wait(CPW - LAG + j, (CPW - LAG + j) % NB)

    @pl.when(w < NTAIL)
    def _():
        pltpu.async_copy(tab.at[src_v.at[CPW]], rows_v.at[0], gsem.at[0]).wait()
        pltpu.sync_copy(rows_v.at[0], acc.at[dst_v.at[CPW]], add=True)


def _load_idx(src_hbm, dst_hbm, src_v, dst_v, w):
    pltpu.sync_copy(src_hbm.at[pl.ds(w * CPW, CPW)], src_v.at[pl.ds(0, CPW)])
    pltpu.sync_copy(dst_hbm.at[pl.ds(w * CPW, CPW)], dst_v.at[pl.ds(0, CPW)])

    @pl.when(w < NTAIL)
    def _():
        pltpu.sync_copy(src_hbm.at[pl.ds(NC * NS * CPW + w, 1)],
                        src_v.at[pl.ds(CPW, 1)])
        pltpu.sync_copy(dst_hbm.at[pl.ds(NC * NS * CPW + w, 1)],
                        dst_v.at[pl.ds(CPW, 1)])


@functools.partial(
    pl.kernel,
    out_type=jax.ShapeDtypeStruct((NC, N_NODES, F_HID), jnp.float32),
    mesh=_mesh,
    compiler_params=_sc_params,
    scratch_types=[
        pltpu.VMEM((CPW + 1, CH), jnp.int32),
        pltpu.VMEM((CPW + 1, CH), jnp.int32),
        pltpu.VMEM((NB, CH, F_HID), jnp.float32),
        pltpu.VMEM_SHARED((N_NODES, F_HID), jnp.float32),
        pltpu.SemaphoreType.DMA((NB,)),
        pltpu.SemaphoreType.DMA((NB,)),
    ],
)
def _sc_agg16(tab_hbm, src_hbm, dst_hbm, zeros_hbm, out_hbm,
              src_v, dst_v, rows_v, acc, gsem, ssem):
    c = lax.axis_index("c")
    s = lax.axis_index("s")
    w = c * NS + s
    rows = pl.ds(s * RPA, RPA)
    pltpu.sync_copy(zeros_hbm.at[rows], acc.at[rows])
    _load_idx(src_hbm, dst_hbm, src_v, dst_v, w)
    plsc.subcore_barrier()
    _agg_pipeline(tab_hbm, acc, src_v, dst_v, rows_v, gsem, ssem, w)
    plsc.subcore_barrier()
    pltpu.sync_copy(acc.at[rows], out_hbm.at[c, rows])


@functools.partial(
    pl.kernel,
    out_type=(
        jax.ShapeDtypeStruct((NC, N_NODES, F_HID), jnp.float32),
        jax.ShapeDtypeStruct((NC, N_NODES, F_HID), jnp.float32),
    ),
    mesh=_mesh,
    compiler_params=_sc_params,
    scratch_types=[
        pltpu.VMEM((RPA, F_HID), jnp.float32),
        pltpu.VMEM((RPA, F_HID), jnp.float32),
        pltpu.VMEM((RPA, F_HID), jnp.float32),
        pltpu.VMEM((RPA, F_HID), jnp.float32),
        pltpu.VMEM((F_HID,), jnp.float32),
        pltpu.VMEM((CPW + 1, CH), jnp.int32),
        pltpu.VMEM((CPW + 1, CH), jnp.int32),
        pltpu.VMEM((NB, CH, F_HID), jnp.float32),
        pltpu.VMEM_SHARED((N_NODES, F_HID), jnp.float32),
        pltpu.SemaphoreType.DMA((NB,)),
        pltpu.SemaphoreType.DMA((NB,)),
    ],
)
def _sc_agg_fused(h1p_hbm, p1_hbm, dinv16_hbm, b1_hbm, src_hbm, dst_hbm,
                  zeros_hbm, y_hbm, out_hbm,
                  hb, pa, pb, dv, b1v, src_v, dst_v, rows_v, acc, gsem, ssem):
    c = lax.axis_index("c")
    s = lax.axis_index("s")
    w = c * NS + s
    rows = pl.ds(s * RPA, RPA)
    pltpu.sync_copy(zeros_hbm.at[rows], acc.at[rows])
    pltpu.sync_copy(h1p_hbm.at[rows], hb)
    pltpu.sync_copy(p1_hbm.at[0, rows], pa)
    pltpu.sync_copy(p1_hbm.at[1, rows], pb)
    pltpu.sync_copy(dinv16_hbm.at[rows], dv)
    pltpu.sync_copy(b1_hbm, b1v)
    _load_idx(src_hbm, dst_hbm, src_v, dst_v, w)
    bv = b1v[...]

    def yrow(i, carry):
        t = (hb[i, :] + pa[i, :] + pb[i, :]) * dv[i, :] + bv
        hb[i, :] = jnp.maximum(t, 0.0) * dv[i, :]
        return carry

    lax.fori_loop(0, RPA, yrow, 0)
    pltpu.sync_copy(hb, y_hbm.at[c, rows])
    plsc.subcore_barrier()
    _agg_pipeline(y_hbm.at[c], acc, src_v, dst_v, rows_v, gsem, ssem, w)
    plsc.subcore_barrier()
    pltpu.sync_copy(acc.at[rows], out_hbm.at[c, rows])


_BLK = 1000
_GRID = N_NODES // _BLK


def _tc1a_body(x_ref, w1_ref, h1_ref):
    h1_ref[...] = jnp.dot(x_ref[...], w1_ref[...],
                          preferred_element_type=jnp.float32)


def _tc1a(x, W1):
    return pl.pallas_call(
        _tc1a_body,
        grid=(_GRID,),
        in_specs=[
            pl.BlockSpec((_BLK, F_IN), lambda i: (i, 0)),
            pl.BlockSpec((F_IN, F_HID), lambda i: (0, 0)),
        ],
        out_specs=pl.BlockSpec((_BLK, F_HID), lambda i: (i, 0)),
        out_shape=jax.ShapeDtypeStruct((N_NODES, F_HID), jnp.float32),
    )(x, W1)


def _tc1b_body(h1_ref, pd0_ref, pd1_ref, h1p_ref, dinv_ref, dinv16_ref):
    deg = pd0_ref[...] + pd1_ref[...] + 1.0
    dinv = lax.rsqrt(deg)
    h1p_ref[...] = h1_ref[...] * dinv
    dinv_ref[...] = dinv
    dinv16_ref[...] = jnp.broadcast_to(dinv, (_BLK, F_HID))


def _tc1b(h1, pd0, pd1):
    return pl.pallas_call(
        _tc1b_body,
        grid=(_GRID,),
        in_specs=[
            pl.BlockSpec((_BLK, F_HID), lambda i: (i, 0)),
            pl.BlockSpec((_BLK, 1), lambda i: (i, 0)),
            pl.BlockSpec((_BLK, 1), lambda i: (i, 0)),
        ],
        out_specs=[
            pl.BlockSpec((_BLK, F_HID), lambda i: (i, 0)),
            pl.BlockSpec((_BLK, 1), lambda i: (i, 0)),
            pl.BlockSpec((_BLK, F_HID), lambda i: (i, 0)),
        ],
        out_shape=[
            jax.ShapeDtypeStruct((N_NODES, F_HID), jnp.float32),
            jax.ShapeDtypeStruct((N_NODES, 1), jnp.float32),
            jax.ShapeDtypeStruct((N_NODES, F_HID), jnp.float32),
        ],
    )(h1, pd0, pd1)


def _tc3_body(y_ref, p0_ref, p1_ref, dinv_ref, b2_ref, w2_ref, out_ref):
    t = y_ref[0] + p0_ref[0] + p1_ref[0]
    h2 = jnp.dot(t, w2_ref[...], preferred_element_type=jnp.float32)
    out_ref[...] = h2 * dinv_ref[...] + b2_ref[...]


def _tc3(y2, p2, dinv, b2, W2):
    return pl.pallas_call(
        _tc3_body,
        grid=(_GRID,),
        in_specs=[
            pl.BlockSpec((1, _BLK, F_HID), lambda i: (0, i, 0)),
            pl.BlockSpec((1, _BLK, F_HID), lambda i: (0, i, 0)),
            pl.BlockSpec((1, _BLK, F_HID), lambda i: (1, i, 0)),
            pl.BlockSpec((_BLK, 1), lambda i: (i, 0)),
            pl.BlockSpec((1, F_EMB), lambda i: (0, 0)),
            pl.BlockSpec((F_HID, F_EMB), lambda i: (0, 0)),
        ],
        out_specs=pl.BlockSpec((_BLK, F_EMB), lambda i: (i, 0)),
        out_shape=jax.ShapeDtypeStruct((N_NODES, F_EMB), jnp.float32),
    )(y2, p2, p2, dinv, b2, W2)


def kernel(x, edge_index, W1, b1, W2, b2):
    src = edge_index[0].reshape(NCHUNK, CH)
    dst = edge_index[1].reshape(NCHUNK, CH)
    z1 = jnp.zeros((N_PAD,), jnp.float32)
    z16 = jnp.zeros((N_NODES, F_HID), jnp.float32)

    h1 = _tc1a(x, W1)
    pd = _sc_deg(dst, z1)
    h1p, dinv, dinv16 = _tc1b(h1, pd[0][:, None], pd[1][:, None])
    p1 = _sc_agg16(h1p, src, dst, z16)
    y2, p2 = _sc_agg_fused(h1p, p1, dinv16, b1, src, dst, z16)
    return _tc3(y2, p2, dinv, b2.reshape(1, F_EMB), W2)


# speedup vs baseline: 1.1598x; 1.0528x over previous
import functools

import jax
import jax.numpy as jnp
from jax import lax
from jax.experimental import pallas as pl
from jax.experimental.pallas import tpu as pltpu
from jax.experimental.pallas import tpu_sc as plsc

N_NODES = 10000
N_PAD = 10240
N_EDGES = 320000
F_IN = 128
F_HID = 16
F_EMB = 32

NC = 2
NS = 16
CH = 128
NCHUNK = N_EDGES // CH
CPW = NCHUNK // (NC * NS)
NTAIL = NCHUNK - CPW * NC * NS
NB = 8
LAG = 4
RPD = N_PAD // NS
RPA = N_NODES // NS

_mesh = plsc.VectorSubcoreMesh(core_axis_name="c", subcore_axis_name="s")
_sc_params = pltpu.CompilerParams(use_tc_tiling_on_sc=False)


@functools.partial(
    pl.kernel,
    out_type=jax.ShapeDtypeStruct((NC, N_PAD), jnp.float32),
    mesh=_mesh,
    compiler_params=_sc_params,
    scratch_types=[
        pltpu.VMEM((CPW + 1, CH), jnp.int32),
        pltpu.VMEM((CH,), jnp.float32),
        pltpu.VMEM_SHARED((N_PAD,), jnp.float32),
        pltpu.SemaphoreType.DMA,
    ],
)
def _sc_deg(dst_hbm, zeros_hbm, out_hbm, dst_v, ones_v, acc, sem):
    c = lax.axis_index("c")
    s = lax.axis_index("s")
    w = c * NS + s
    for k in range(CH // 16):
        ones_v[pl.ds(k * 16, 16)] = jnp.ones((16,), jnp.float32)
    pltpu.sync_copy(zeros_hbm.at[pl.ds(s * RPD, RPD)], acc.at[pl.ds(s * RPD, RPD)])
    pltpu.sync_copy(dst_hbm.at[pl.ds(w * CPW, CPW)], dst_v.at[pl.ds(0, CPW)])

    @pl.when(w < NTAIL)
    def _():
        pltpu.sync_copy(dst_hbm.at[pl.ds(NC * NS * CPW + w, 1)],
                        dst_v.at[pl.ds(CPW, 1)])

    plsc.subcore_barrier()

    def fire(i, carry):
        pltpu.async_copy(ones_v, acc.at[dst_v.at[i]], sem, add=True)
        return carry

    def drain(i, carry):
        pltpu.make_async_copy(ones_v, acc.at[dst_v.at[i]], sem).wait()
        return carry

    n = CPW + jnp.where(w < NTAIL, 1, 0)
    lax.fori_loop(0, n, fire, 0)
    lax.fori_loop(0, n, drain, 0)
    plsc.subcore_barrier()
    pltpu.sync_copy(acc.at[pl.ds(s * RPD, RPD)], out_hbm.at[c, pl.ds(s * RPD, RPD)])


def _agg_pipeline(tab, acc, src_v, dst_v, rows_v, gsem, ssem, w):
    def g_start(i, b):
        pltpu.async_copy(tab.at[src_v.at[i]], rows_v.at[b], gsem.at[b])

    def g_wait(i, b):
        pltpu.make_async_copy(tab.at[src_v.at[i]], rows_v.at[b],
                              gsem.at[b]).wait()

    def s_start(i, b):
        pltpu.async_copy(rows_v.at[b], acc.at[dst_v.at[i]], ssem.at[b],
                         add=True)

    def s_wait(i, b):
        pltpu.make_async_copy(rows_v.at[b], acc.at[dst_v.at[i]],
                              ssem.at[b]).wait()

    for j in range(LAG):
        g_start(j, j)

    def chunk(i, carry):
        b = lax.rem(i, NB)
        g_wait(i, b)
        s_start(i, b)

        @pl.when(i >= LAG)
        def _():
            s_wait(i - LAG, lax.rem(i - LAG, NB))

        @pl.when(i + LAG < CPW)
        def _():
            g_start(i + LAG, lax.rem(i + LAG, NB))

        return carry

    lax.fori_loop(0, CPW, chunk, 0)
    for j in range(LAG):
        s_wait(CPW - LAG + j, (CPW - LAG + j) % NB)

    @pl.when(w < NTAIL)
    def _():
        pltpu.async_copy(tab.at[src_v.at[CPW]], rows_v.at[0], gsem.at[0]).wait()
        pltpu.sync_copy(rows_v.at[0], acc.at[dst_v.at[CPW]], add=True)


def _load_idx(src_hbm, dst_hbm, src_v, dst_v, w):
    pltpu.sync_copy(src_hbm.at[pl.ds(w * CPW, CPW)], src_v.at[pl.ds(0, CPW)])
    pltpu.sync_copy(dst_hbm.at[pl.ds(w * CPW, CPW)], dst_v.at[pl.ds(0, CPW)])

    @pl.when(w < NTAIL)
    def _():
        pltpu.sync_copy(src_hbm.at[pl.ds(NC * NS * CPW + w, 1)],
                        src_v.at[pl.ds(CPW, 1)])
        pltpu.sync_copy(dst_hbm.at[pl.ds(NC * NS * CPW + w, 1)],
                        dst_v.at[pl.ds(CPW, 1)])


@functools.partial(
    pl.kernel,
    out_type=jax.ShapeDtypeStruct((NC, N_NODES, F_HID), jnp.float32),
    mesh=_mesh,
    compiler_params=_sc_params,
    scratch_types=[
        pltpu.VMEM((CPW + 1, CH), jnp.int32),
        pltpu.VMEM((CPW + 1, CH), jnp.int32),
        pltpu.VMEM((NB, CH, F_HID), jnp.float32),
        pltpu.VMEM_SHARED((N_NODES, F_HID), jnp.float32),
        pltpu.SemaphoreType.DMA((NB,)),
        pltpu.SemaphoreType.DMA((NB,)),
    ],
)
def _sc_agg16(tab_hbm, src_hbm, dst_hbm, zeros_hbm, out_hbm,
              src_v, dst_v, rows_v, acc, gsem, ssem):
    c = lax.axis_index("c")
    s = lax.axis_index("s")
    w = c * NS + s
    rows = pl.ds(s * RPA, RPA)
    pltpu.sync_copy(zeros_hbm.at[rows], acc.at[rows])
    _load_idx(src_hbm, dst_hbm, src_v, dst_v, w)
    plsc.subcore_barrier()
    _agg_pipeline(tab_hbm, acc, src_v, dst_v, rows_v, gsem, ssem, w)
    plsc.subcore_barrier()
    pltpu.sync_copy(acc.at[rows], out_hbm.at[c, rows])


@functools.partial(
    pl.kernel,
    out_type=(
        jax.ShapeDtypeStruct((NC, N_NODES, F_HID), jnp.float32),
        jax.ShapeDtypeStruct((NC, N_NODES, F_HID), jnp.float32),
    ),
    mesh=_mesh,
    compiler_params=_sc_params,
    scratch_types=[
        pltpu.VMEM((RPA, F_HID), jnp.float32),
        pltpu.VMEM((RPA, F_HID), jnp.float32),
        pltpu.VMEM((RPA, F_HID), jnp.float32),
        pltpu.VMEM((RPA, F_HID), jnp.float32),
        pltpu.VMEM((F_HID,), jnp.float32),
        pltpu.VMEM((CPW + 1, CH), jnp.int32),
        pltpu.VMEM((CPW + 1, CH), jnp.int32),
        pltpu.VMEM((NB, CH, F_HID), jnp.float32),
        pltpu.VMEM_SHARED((N_NODES, F_HID), jnp.float32),
        pltpu.SemaphoreType.DMA((NB,)),
        pltpu.SemaphoreType.DMA((NB,)),
    ],
)
def _sc_agg_fused(h1p_hbm, p1_hbm, dinv16_hbm, b1_hbm, src_hbm, dst_hbm,
                  zeros_hbm, y_hbm, out_hbm,
                  hb, pa, pb, dv, b1v, src_v, dst_v, rows_v, acc, gsem, ssem):
    c = lax.axis_index("c")
    s = lax.axis_index("s")
    w = c * NS + s
    rows = pl.ds(s * RPA, RPA)
    pltpu.sync_copy(zeros_hbm.at[rows], acc.at[rows])
    pltpu.sync_copy(h1p_hbm.at[rows], hb)
    pltpu.sync_copy(p1_hbm.at[0, rows], pa)
    pltpu.sync_copy(p1_hbm.at[1, rows], pb)
    pltpu.sync_copy(dinv16_hbm.at[rows], dv)
    pltpu.sync_copy(b1_hbm, b1v)
    _load_idx(src_hbm, dst_hbm, src_v, dst_v, w)
    bv = b1v[...]

    def yrow(i, carry):
        t = (hb[i, :] + pa[i, :] + pb[i, :]) * dv[i, :] + bv
        hb[i, :] = jnp.maximum(t, 0.0) * dv[i, :]
        return carry

    lax.fori_loop(0, RPA, yrow, 0)
    pltpu.sync_copy(hb, y_hbm.at[c, rows])
    plsc.subcore_barrier()
    _agg_pipeline(y_hbm.at[c], acc, src_v, dst_v, rows_v, gsem, ssem, w)
    plsc.subcore_barrier()
    pltpu.sync_copy(acc.at[rows], out_hbm.at[c, rows])


_BLK = 1000
_GRID = N_NODES // _BLK


def _tc1a_body(x_ref, w1_ref, h1_ref):
    h1_ref[...] = jnp.dot(x_ref[...], w1_ref[...],
                          preferred_element_type=jnp.float32)


def _tc1a(x, W1):
    return pl.pallas_call(
        _tc1a_body,
        grid=(_GRID,),
        in_specs=[
            pl.BlockSpec((_BLK, F_IN), lambda i: (i, 0)),
            pl.BlockSpec((F_IN, F_HID), lambda i: (0, 0)),
        ],
        out_specs=pl.BlockSpec((_BLK, F_HID), lambda i: (i, 0)),
        out_shape=jax.ShapeDtypeStruct((N_NODES, F_HID), jnp.float32),
    )(x, W1)


def _tc1b_body(h1_ref, pd0_ref, pd1_ref, h1p_ref, dinv_ref, dinv16_ref):
    deg = pd0_ref[...] + pd1_ref[...] + 1.0
    dinv = lax.rsqrt(deg)
    h1p_ref[...] = h1_ref[...] * dinv
    dinv_ref[...] = dinv
    dinv16_ref[...] = jnp.broadcast_to(dinv, (_BLK, F_HID))


def _tc1b(h1, pd0, pd1):
    return pl.pallas_call(
        _tc1b_body,
        grid=(_GRID,),
        in_specs=[
            pl.BlockSpec((_BLK, F_HID), lambda i: (i, 0)),
            pl.BlockSpec((_BLK, 1), lambda i: (i, 0)),
            pl.BlockSpec((_BLK, 1), lambda i: (i, 0)),
        ],
        out_specs=[
            pl.BlockSpec((_BLK, F_HID), lambda i: (i, 0)),
            pl.BlockSpec((_BLK, 1), lambda i: (i, 0)),
            pl.BlockSpec((_BLK, F_HID), lambda i: (i, 0)),
        ],
        out_shape=[
            jax.ShapeDtypeStruct((N_NODES, F_HID), jnp.float32),
            jax.ShapeDtypeStruct((N_NODES, 1), jnp.float32),
            jax.ShapeDtypeStruct((N_NODES, F_HID), jnp.float32),
        ],
    )(h1, pd0, pd1)


def _tc3_body(y_ref, p0_ref, p1_ref, dinv_ref, b2_ref, w2_ref, out_ref):
    t = y_ref[0] + p0_ref[0] + p1_ref[0]
    h2 = jnp.dot(t, w2_ref[...], preferred_element_type=jnp.float32)
    out_ref[...] = h2 * dinv_ref[...] + b2_ref[...]


def _tc3(y2, p2, dinv, b2, W2):
    return pl.pallas_call(
        _tc3_body,
        grid=(_GRID,),
        in_specs=[
            pl.BlockSpec((1, _BLK, F_HID), lambda i: (0, i, 0)),
            pl.BlockSpec((1, _BLK, F_HID), lambda i: (0, i, 0)),
            pl.BlockSpec((1, _BLK, F_HID), lambda i: (1, i, 0)),
            pl.BlockSpec((_BLK, 1), lambda i: (i, 0)),
            pl.BlockSpec((1, F_EMB), lambda i: (0, 0)),
            pl.BlockSpec((F_HID, F_EMB), lambda i: (0, 0)),
        ],
        out_specs=pl.BlockSpec((_BLK, F_EMB), lambda i: (i, 0)),
        out_shape=jax.ShapeDtypeStruct((N_NODES, F_EMB), jnp.float32),
    )(y2, p2, p2, dinv, b2, W2)


def kernel(x, edge_index, W1, b1, W2, b2):
    src = edge_index[0].reshape(NCHUNK, CH)
    dst = edge_index[1].reshape(NCHUNK, CH)
    z1 = jnp.zeros((N_PAD,), jnp.float32)
    z16 = jnp.zeros((N_NODES, F_HID), jnp.float32)

    h1 = _tc1a(x, W1)
    pd = _sc_deg(dst, z1)
    h1p, dinv, dinv16 = _tc1b(h1, pd[0][:, None], pd[1][:, None])
    p1 = _sc_agg16(h1p, src, dst, z16)
    y2, p2 = _sc_agg_fused(h1p, p1, dinv16, b1, src, dst, z16)
    return _tc3(y2, p2, dinv, b2.reshape(1, F_EMB), W2)
